# Initial kernel scaffold; baseline (speedup 1.0000x reference)
import jax
import jax.numpy as jnp
from jax.experimental import pallas as pl


def kernel(features, W1, b1, W2, b2, W3, b3):
    raise NotImplementedError("write your pallas kernel here")


# speedup vs baseline: 2.0008x; 2.0008x over previous
import jax
import jax.numpy as jnp
from jax.experimental import pallas as pl
from jax.experimental.pallas import tpu as pltpu

_NUM_EXPERTS = 16
_BLOCK_M = 2048


def _fused_gating_kernel(x_ref, w1_ref, b1_ref, w2_ref, b2_ref, w3_ref, b3_ref,
                         out_ref):
    x = x_ref[...]
    h = jnp.maximum(jnp.dot(x, w1_ref[...]) + b1_ref[...], 0.0)
    h = jnp.maximum(jnp.dot(h, w2_ref[...]) + b2_ref[...], 0.0)
    logits = jnp.dot(h, w3_ref[...]) + b3_ref[...]
    m = jnp.max(logits, axis=1, keepdims=True)
    col = jax.lax.broadcasted_iota(jnp.int32, logits.shape, 1)
    idx = jnp.min(jnp.where(logits == m, col, _NUM_EXPERTS), axis=1,
                  keepdims=True)
    out_ref[...] = (col == idx).astype(jnp.float32)


def kernel(features, W1, b1, W2, b2, W3, b3):
    n_tokens, input_size = features.shape
    hidden = W1.shape[0]
    hidden2 = W2.shape[0]
    n_experts = W3.shape[0]

    w1t = W1.T
    w2t = W2.T
    w3t = W3.T
    b1r = b1.reshape(1, hidden)
    b2r = b2.reshape(1, hidden2)
    b3r = b3.reshape(1, n_experts)

    bm = min(_BLOCK_M, n_tokens)
    grid = (n_tokens // bm,)

    return pl.pallas_call(
        _fused_gating_kernel,
        grid=grid,
        in_specs=[
            pl.BlockSpec((bm, input_size), lambda i: (i, 0)),
            pl.BlockSpec((input_size, hidden), lambda i: (0, 0)),
            pl.BlockSpec((1, hidden), lambda i: (0, 0)),
            pl.BlockSpec((hidden, hidden2), lambda i: (0, 0)),
            pl.BlockSpec((1, hidden2), lambda i: (0, 0)),
            pl.BlockSpec((hidden2, n_experts), lambda i: (0, 0)),
            pl.BlockSpec((1, n_experts), lambda i: (0, 0)),
        ],
        out_specs=pl.BlockSpec((bm, n_experts), lambda i: (i, 0)),
        out_shape=jax.ShapeDtypeStruct((n_tokens, n_experts), jnp.float32),
        compiler_params=pltpu.CompilerParams(
            dimension_semantics=("arbitrary",),
        ),
    )(features, w1t, b1r, w2t, b2r, w3t, b3r)


# speedup vs baseline: 2.0018x; 1.0005x over previous
import jax
import jax.numpy as jnp
from jax.experimental import pallas as pl
from jax.experimental.pallas import tpu as pltpu

_NUM_EXPERTS = 16
_BLOCK_M = 2048


def _fused_gating_kernel(x_ref, w1_ref, b1_ref, w2_ref, b2_ref, w3_ref, b3_ref,
                         out_ref):
    x = x_ref[...]
    h = jnp.maximum(jnp.dot(x, w1_ref[...]) + b1_ref[...], 0.0)
    h = jnp.maximum(jnp.dot(h, w2_ref[...]) + b2_ref[...], 0.0)
    logits = jnp.dot(h, w3_ref[...]) + b3_ref[...]
    m = jnp.max(logits, axis=1, keepdims=True)
    col = jax.lax.broadcasted_iota(jnp.int32, logits.shape, 1)
    idx = jnp.min(jnp.where(logits == m, col, _NUM_EXPERTS), axis=1,
                  keepdims=True)
    out_ref[...] = (col == idx).astype(jnp.float32)


def kernel(features, W1, b1, W2, b2, W3, b3):
    n_tokens, input_size = features.shape
    hidden = W1.shape[0]
    hidden2 = W2.shape[0]
    n_experts = W3.shape[0]

    w1t = W1.T
    w2t = W2.T
    w3t = W3.T
    b1r = b1.reshape(1, hidden)
    b2r = b2.reshape(1, hidden2)
    b3r = b3.reshape(1, n_experts)

    bm = min(_BLOCK_M, n_tokens)
    grid = (n_tokens // bm,)

    return pl.pallas_call(
        _fused_gating_kernel,
        grid=grid,
        in_specs=[
            pl.BlockSpec((bm, input_size), lambda i: (i, 0)),
            pl.BlockSpec((input_size, hidden), lambda i: (0, 0)),
            pl.BlockSpec((1, hidden), lambda i: (0, 0)),
            pl.BlockSpec((hidden, hidden2), lambda i: (0, 0)),
            pl.BlockSpec((1, hidden2), lambda i: (0, 0)),
            pl.BlockSpec((hidden2, n_experts), lambda i: (0, 0)),
            pl.BlockSpec((1, n_experts), lambda i: (0, 0)),
        ],
        out_specs=pl.BlockSpec((bm, n_experts), lambda i: (i, 0)),
        out_shape=jax.ShapeDtypeStruct((n_tokens, n_experts), jnp.float32),
        compiler_params=pltpu.CompilerParams(
            dimension_semantics=("parallel",),
        ),
    )(features, w1t, b1r, w2t, b2r, w3t, b3r)


# speedup vs baseline: 2.2921x; 1.1450x over previous
import jax
import jax.numpy as jnp
from jax.experimental import pallas as pl
from jax.experimental.pallas import tpu as pltpu

_NUM_EXPERTS = 16
_BLOCK_M = 2048

_DNT = (((1,), (1,)), ((), ()))


def _fused_gating_kernel(x_ref, w1_ref, b1_ref, w2_ref, b2_ref, w3_ref, b3_ref,
                         out_ref):
    x = x_ref[...]
    h = jnp.maximum(jax.lax.dot_general(x, w1_ref[...], _DNT) + b1_ref[...],
                    0.0)
    h = jnp.maximum(jax.lax.dot_general(h, w2_ref[...], _DNT) + b2_ref[...],
                    0.0)
    logits = jax.lax.dot_general(h, w3_ref[...], _DNT) + b3_ref[...]
    m = jnp.max(logits, axis=1, keepdims=True)
    col = jax.lax.broadcasted_iota(jnp.int32, logits.shape, 1)
    idx = jnp.min(jnp.where(logits == m, col, _NUM_EXPERTS), axis=1,
                  keepdims=True)
    out_ref[...] = (col == idx).astype(jnp.float32)


def kernel(features, W1, b1, W2, b2, W3, b3):
    n_tokens, input_size = features.shape
    hidden = W1.shape[0]
    hidden2 = W2.shape[0]
    n_experts = W3.shape[0]

    b1r = b1.reshape(1, hidden)
    b2r = b2.reshape(1, hidden2)
    b3r = b3.reshape(1, n_experts)

    bm = min(_BLOCK_M, n_tokens)
    grid = (n_tokens // bm,)

    return pl.pallas_call(
        _fused_gating_kernel,
        grid=grid,
        in_specs=[
            pl.BlockSpec((bm, input_size), lambda i: (i, 0)),
            pl.BlockSpec((hidden, input_size), lambda i: (0, 0)),
            pl.BlockSpec((1, hidden), lambda i: (0, 0)),
            pl.BlockSpec((hidden2, hidden), lambda i: (0, 0)),
            pl.BlockSpec((1, hidden2), lambda i: (0, 0)),
            pl.BlockSpec((n_experts, hidden2), lambda i: (0, 0)),
            pl.BlockSpec((1, n_experts), lambda i: (0, 0)),
        ],
        out_specs=pl.BlockSpec((bm, n_experts), lambda i: (i, 0)),
        out_shape=jax.ShapeDtypeStruct((n_tokens, n_experts), jnp.float32),
        compiler_params=pltpu.CompilerParams(
            dimension_semantics=("parallel",),
        ),
    )(features, W1, b1r, W2, b2r, W3, b3r)
